# baseline (speedup 1.0000x reference)
import functools

import jax
import jax.numpy as jnp
from jax import lax
from jax.experimental import pallas as pl
from jax.experimental.pallas import tpu as pltpu
from jax.experimental.pallas import tpu_sc as plsc

B = 8
L = 2048
D = 512
TOTAL = B * L // 2
NROWS = B * L

NC = 2
NS = 16
NW = NC * NS
LANES = 16

TPW = TOTAL // NW
CH = 64
NCH = TPW // CH
ZROWS = 8

_mesh = plsc.VectorSubcoreMesh(core_axis_name="c", subcore_axis_name="s")


@functools.partial(
    pl.kernel,
    out_type=jax.ShapeDtypeStruct((NROWS, D), jnp.float32),
    mesh=_mesh,
    scratch_types=[
        pltpu.VMEM((16,), jnp.int32),
        pltpu.VMEM((NCH, CH), jnp.int32),
        pltpu.VMEM((NCH, CH), jnp.int32),
        pltpu.VMEM((2, CH, D), jnp.float32),
        pltpu.VMEM((CH, D), jnp.float32),
        pltpu.SemaphoreType.DMA,
        pltpu.SemaphoreType.DMA,
        pltpu.SemaphoreType.DMA,
        pltpu.SemaphoreType.DMA,
        pltpu.SemaphoreType.DMA,
        pltpu.SemaphoreType.DMA,
        pltpu.SemaphoreType.DMA,
    ],
    compiler_params=pltpu.CompilerParams(needs_layout_passes=False),
)
def _to_dense(flat_hbm, cu_hbm, zeros_hbm, out_hbm,
              cu_v, didx, pidx, dbuf, zbuf,
              sem_cu, sem_z, sem_in0, sem_in1, sem_sc0, sem_sc1, sem_p):
    wid = lax.axis_index("s") * NC + lax.axis_index("c")
    base = pl.multiple_of(wid * TPW, TPW)
    sem_in = (sem_in0, sem_in1)
    sem_sc = (sem_sc0, sem_sc1)

    def load(c):
        return pltpu.async_copy(
            flat_hbm.at[pl.ds(base + c * CH, CH)], dbuf.at[c % 2],
            sem_in[c % 2])

    cp_cu = pltpu.async_copy(cu_hbm, cu_v, sem_cu)
    cp_zs = [pltpu.async_copy(zeros_hbm, zbuf.at[pl.ds(i * ZROWS, ZROWS)],
                              sem_z)
             for i in range(CH // ZROWS)]
    lds = {0: load(0), 1: load(1)}
    cp_cu.wait()

    iota = lax.iota(jnp.int32, LANES)
    cuvec = cu_v[...]
    cus = [None] + [jnp.max(jnp.where(iota == k, cuvec, 0))
                    for k in range(1, B + 1)]
    cub = [plsc.load_gather(cu_v, [jnp.full((LANES,), k, jnp.int32)])
           for k in range(1, B + 1)]

    def sel(idx, vals):
        acc = jnp.int32(0)
        for k in range(1, len(vals)):
            acc = jnp.where(idx == k, vals[k], acc)
        return acc

    def write_data(c):
        t0 = base + c * CH
        r0 = jnp.int32(0)
        r1 = jnp.int32(0)
        for k in range(1, B + 1):
            r0 = r0 + (t0 >= cus[k]).astype(jnp.int32)
            r1 = r1 + (t0 + (CH - 1) >= cus[k]).astype(jnp.int32)
        d0 = r0 * L + t0 - sel(r0, [None] + cus[1:])
        contig = jnp.logical_and(r0 == r1, d0 % 8 == 0)

        @pl.when(contig)
        def _():
            d0a = pl.multiple_of(d0, 8)
            pltpu.async_copy(dbuf.at[c % 2], out_hbm.at[pl.ds(d0a, CH)],
                             sem_sc[c % 2])

        @pl.when(jnp.logical_not(contig))
        def _():
            for j in range(CH // LANES):
                ivj = (t0 + j * LANES) + iota
                r = jnp.zeros((LANES,), jnp.int32)
                for k in range(B):
                    r = r + (ivj >= cub[k]).astype(jnp.int32)
                didx[c, pl.ds(j * LANES, LANES)] = (
                    r * L + ivj - plsc.load_gather(cu_v, [r]))
            pltpu.async_copy(dbuf.at[c % 2], out_hbm.at[didx.at[c]],
                             sem_sc[c % 2])

    def wait_write(c):
        pltpu.make_async_copy(dbuf.at[c % 2], out_hbm.at[pl.ds(0, CH)],
                              sem_sc[c % 2]).wait()

    def write_pad(c):
        q0 = base + c * CH
        b0 = jnp.int32(0)
        b1 = jnp.int32(0)
        for k in range(1, B + 1):
            pcu_k = k * L - cus[k]
            b0 = b0 + (q0 >= pcu_k).astype(jnp.int32)
            b1 = b1 + (q0 + (CH - 1) >= pcu_k).astype(jnp.int32)
        p0 = q0 + sel(b0 + 1, [None] + cus[1:] + [cus[B]])
        contig = jnp.logical_and(b0 == b1, p0 % 8 == 0)

        @pl.when(contig)
        def _():
            p0a = pl.multiple_of(p0, 8)
            pltpu.async_copy(zbuf, out_hbm.at[pl.ds(p0a, CH)], sem_p)

        @pl.when(jnp.logical_not(contig))
        def _():
            for j in range(CH // LANES):
                qvj = (q0 + j * LANES) + iota
                b = jnp.zeros((LANES,), jnp.int32)
                for k in range(B):
                    b = b + (qvj >= ((k + 1) * L - cub[k])).astype(jnp.int32)
                pidx[c, pl.ds(j * LANES, LANES)] = (
                    plsc.load_gather(cu_v, [b + 1]) + qvj)
            pltpu.async_copy(zbuf, out_hbm.at[pidx.at[c]], sem_p)

    for cp in cp_zs:
        cp.wait()
    for c in range(NCH):
        write_pad(c)

    lds[0].wait()
    write_data(0)
    lds[1].wait()
    write_data(1)
    wait_write(0)
    lds[2] = load(2)
    wait_write(1)
    lds[3] = load(3)
    lds[2].wait()
    write_data(2)
    lds[3].wait()
    write_data(3)
    wait_write(2)
    wait_write(3)
    for _ in range(NCH):
        pltpu.make_async_copy(zbuf, out_hbm.at[pl.ds(0, CH)], sem_p).wait()


def kernel(flat, cu_seqlens):
    cu_pad = jnp.zeros((16,), jnp.int32).at[:B + 1].set(
        cu_seqlens.astype(jnp.int32))
    zeros = jnp.zeros((ZROWS, D), jnp.float32)
    return _to_dense(flat, cu_pad, zeros).reshape(B, L, D)


# speedup vs baseline: 1.1686x; 1.1686x over previous
import functools

import jax
import jax.numpy as jnp
from jax import lax
from jax.experimental import pallas as pl
from jax.experimental.pallas import tpu as pltpu
from jax.experimental.pallas import tpu_sc as plsc

B = 8
L = 2048
D = 512
TOTAL = B * L // 2
NROWS = B * L

NC = 2
NS = 16
NW = NC * NS
LANES = 16

TPW = TOTAL // NW
CH = 64
NCH = TPW // CH

_mesh = plsc.VectorSubcoreMesh(core_axis_name="c", subcore_axis_name="s")


@functools.partial(
    pl.kernel,
    out_type=jax.ShapeDtypeStruct((NROWS, D), jnp.float32),
    mesh=_mesh,
    scratch_types=[
        pltpu.VMEM((16,), jnp.int32),
        pltpu.VMEM((NCH, CH), jnp.int32),
        pltpu.VMEM((NCH, CH), jnp.int32),
        pltpu.VMEM((2, CH, D), jnp.float32),
        pltpu.VMEM((CH, D), jnp.float32),
        pltpu.SemaphoreType.DMA,
        pltpu.SemaphoreType.DMA,
        pltpu.SemaphoreType.DMA,
        pltpu.SemaphoreType.DMA,
        pltpu.SemaphoreType.DMA,
        pltpu.SemaphoreType.DMA,
        pltpu.SemaphoreType.DMA,
    ],
    compiler_params=pltpu.CompilerParams(needs_layout_passes=False,
                                         skip_device_barrier=True),
)
def _to_dense(flat_hbm, cu_hbm, zeros_hbm, out_hbm,
              cu_v, didx, pidx, dbuf, zbuf,
              sem_cu, sem_z, sem_in0, sem_in1, sem_sc0, sem_sc1, sem_p):
    wid = lax.axis_index("s") * NC + lax.axis_index("c")
    base = pl.multiple_of(wid * TPW, TPW)
    sem_in = (sem_in0, sem_in1)
    sem_sc = (sem_sc0, sem_sc1)

    def load(c):
        return pltpu.async_copy(
            flat_hbm.at[pl.ds(base + c * CH, CH)], dbuf.at[c % 2],
            sem_in[c % 2])

    cp_cu = pltpu.async_copy(cu_hbm, cu_v, sem_cu)
    cp_z = pltpu.async_copy(zeros_hbm, zbuf, sem_z)
    lds = {0: load(0), 1: load(1)}
    cp_cu.wait()

    iota = lax.iota(jnp.int32, LANES)
    cuvec = cu_v[...]
    cus = [None] + [jnp.max(jnp.where(iota == k, cuvec, 0))
                    for k in range(1, B + 1)]
    cub = [plsc.load_gather(cu_v, [jnp.full((LANES,), k, jnp.int32)])
           for k in range(1, B + 1)]

    def sel(idx, vals):
        acc = jnp.int32(0)
        for k in range(1, len(vals)):
            acc = jnp.where(idx == k, vals[k], acc)
        return acc

    def write_data(c):
        t0 = base + c * CH
        r0 = jnp.int32(0)
        r1 = jnp.int32(0)
        for k in range(1, B + 1):
            r0 = r0 + (t0 >= cus[k]).astype(jnp.int32)
            r1 = r1 + (t0 + (CH - 1) >= cus[k]).astype(jnp.int32)
        d0 = r0 * L + t0 - sel(r0, [None] + cus[1:])
        contig = jnp.logical_and(r0 == r1, d0 % 8 == 0)

        @pl.when(contig)
        def _():
            d0a = pl.multiple_of(d0, 8)
            pltpu.async_copy(dbuf.at[c % 2], out_hbm.at[pl.ds(d0a, CH)],
                             sem_sc[c % 2])

        @pl.when(jnp.logical_not(contig))
        def _():
            for j in range(CH // LANES):
                ivj = (t0 + j * LANES) + iota
                r = jnp.zeros((LANES,), jnp.int32)
                for k in range(B):
                    r = r + (ivj >= cub[k]).astype(jnp.int32)
                didx[c, pl.ds(j * LANES, LANES)] = (
                    r * L + ivj - plsc.load_gather(cu_v, [r]))
            pltpu.async_copy(dbuf.at[c % 2], out_hbm.at[didx.at[c]],
                             sem_sc[c % 2])

    def wait_write(c):
        pltpu.make_async_copy(dbuf.at[c % 2], out_hbm.at[pl.ds(0, CH)],
                              sem_sc[c % 2]).wait()

    def write_pad(c):
        q0 = base + c * CH
        b0 = jnp.int32(0)
        b1 = jnp.int32(0)
        for k in range(1, B + 1):
            pcu_k = k * L - cus[k]
            b0 = b0 + (q0 >= pcu_k).astype(jnp.int32)
            b1 = b1 + (q0 + (CH - 1) >= pcu_k).astype(jnp.int32)
        p0 = q0 + sel(b0 + 1, [None] + cus[1:] + [cus[B]])
        contig = jnp.logical_and(b0 == b1, p0 % 8 == 0)

        @pl.when(contig)
        def _():
            p0a = pl.multiple_of(p0, 8)
            pltpu.async_copy(zbuf, out_hbm.at[pl.ds(p0a, CH)], sem_p)

        @pl.when(jnp.logical_not(contig))
        def _():
            for j in range(CH // LANES):
                qvj = (q0 + j * LANES) + iota
                b = jnp.zeros((LANES,), jnp.int32)
                for k in range(B):
                    b = b + (qvj >= ((k + 1) * L - cub[k])).astype(jnp.int32)
                pidx[c, pl.ds(j * LANES, LANES)] = (
                    plsc.load_gather(cu_v, [b + 1]) + qvj)
            pltpu.async_copy(zbuf, out_hbm.at[pidx.at[c]], sem_p)

    cp_z.wait()
    for c in range(NCH):
        write_pad(c)

    lds[0].wait()
    write_data(0)
    lds[1].wait()
    write_data(1)
    wait_write(0)
    lds[2] = load(2)
    wait_write(1)
    lds[3] = load(3)
    lds[2].wait()
    write_data(2)
    lds[3].wait()
    write_data(3)
    wait_write(2)
    wait_write(3)
    for _ in range(NCH):
        pltpu.make_async_copy(zbuf, out_hbm.at[pl.ds(0, CH)], sem_p).wait()


def kernel(flat, cu_seqlens):
    cu_pad = jnp.zeros((16,), jnp.int32).at[:B + 1].set(
        cu_seqlens.astype(jnp.int32))
    zeros = jnp.zeros((CH, D), jnp.float32)
    return _to_dense(flat, cu_pad, zeros).reshape(B, L, D)


# speedup vs baseline: 1.1783x; 1.0083x over previous
import functools

import jax
import jax.numpy as jnp
from jax import lax
from jax.experimental import pallas as pl
from jax.experimental.pallas import tpu as pltpu
from jax.experimental.pallas import tpu_sc as plsc

B = 8
L = 2048
D = 512
TOTAL = B * L // 2
NROWS = B * L

NC = 2
NS = 16
NW = NC * NS
LANES = 16

TPW = TOTAL // NW
CH = 64
NCH = TPW // CH

_mesh = plsc.VectorSubcoreMesh(core_axis_name="c", subcore_axis_name="s")


@functools.partial(
    pl.kernel,
    out_type=jax.ShapeDtypeStruct((NROWS, D), jnp.float32),
    mesh=_mesh,
    scratch_types=[
        pltpu.VMEM((16,), jnp.int32),
        pltpu.VMEM((NCH, CH), jnp.int32),
        pltpu.VMEM((NCH, CH), jnp.int32),
        pltpu.VMEM((2, CH, D), jnp.float32),
        pltpu.VMEM((CH, D), jnp.float32),
        pltpu.SemaphoreType.DMA,
        pltpu.SemaphoreType.DMA,
        pltpu.SemaphoreType.DMA,
        pltpu.SemaphoreType.DMA,
        pltpu.SemaphoreType.DMA,
        pltpu.SemaphoreType.DMA,
    ],
    compiler_params=pltpu.CompilerParams(needs_layout_passes=False),
)
def _to_dense(flat_hbm, cu_hbm, out_hbm,
              cu_v, didx, pidx, dbuf, zbuf,
              sem_cu, sem_in0, sem_in1, sem_sc0, sem_sc1, sem_p):
    wid = lax.axis_index("s") * NC + lax.axis_index("c")
    base = pl.multiple_of(wid * TPW, TPW)
    sem_in = (sem_in0, sem_in1)
    sem_sc = (sem_sc0, sem_sc1)

    def load(c):
        return pltpu.async_copy(
            flat_hbm.at[pl.ds(base + c * CH, CH)], dbuf.at[c % 2],
            sem_in[c % 2])

    cp_cu = pltpu.async_copy(cu_hbm, cu_v.at[pl.ds(0, B + 1)], sem_cu)
    lds = {0: load(0), 1: load(1)}
    zvec = jnp.zeros((LANES,), jnp.float32)
    for zr in range(CH):
        for zj in range(D // LANES):
            zbuf[zr, pl.ds(zj * LANES, LANES)] = zvec
    cp_cu.wait()

    iota = lax.iota(jnp.int32, LANES)
    cuvec = cu_v[...]
    cus = [None] + [jnp.max(jnp.where(iota == k, cuvec, 0))
                    for k in range(1, B + 1)]
    cub = [plsc.load_gather(cu_v, [jnp.full((LANES,), k, jnp.int32)])
           for k in range(1, B + 1)]

    def sel(idx, vals):
        acc = jnp.int32(0)
        for k in range(1, len(vals)):
            acc = jnp.where(idx == k, vals[k], acc)
        return acc

    def write_data(c):
        t0 = base + c * CH
        r0 = jnp.int32(0)
        r1 = jnp.int32(0)
        for k in range(1, B + 1):
            r0 = r0 + (t0 >= cus[k]).astype(jnp.int32)
            r1 = r1 + (t0 + (CH - 1) >= cus[k]).astype(jnp.int32)
        d0 = r0 * L + t0 - sel(r0, [None] + cus[1:])
        contig = jnp.logical_and(r0 == r1, d0 % 8 == 0)

        @pl.when(contig)
        def _():
            d0a = pl.multiple_of(d0, 8)
            pltpu.async_copy(dbuf.at[c % 2], out_hbm.at[pl.ds(d0a, CH)],
                             sem_sc[c % 2])

        @pl.when(jnp.logical_not(contig))
        def _():
            for j in range(CH // LANES):
                ivj = (t0 + j * LANES) + iota
                r = jnp.zeros((LANES,), jnp.int32)
                for k in range(B):
                    r = r + (ivj >= cub[k]).astype(jnp.int32)
                didx[c, pl.ds(j * LANES, LANES)] = (
                    r * L + ivj - plsc.load_gather(cu_v, [r]))
            pltpu.async_copy(dbuf.at[c % 2], out_hbm.at[didx.at[c]],
                             sem_sc[c % 2])

    def wait_write(c):
        pltpu.make_async_copy(dbuf.at[c % 2], out_hbm.at[pl.ds(0, CH)],
                              sem_sc[c % 2]).wait()

    def write_pad(c):
        q0 = base + c * CH
        b0 = jnp.int32(0)
        b1 = jnp.int32(0)
        for k in range(1, B + 1):
            pcu_k = k * L - cus[k]
            b0 = b0 + (q0 >= pcu_k).astype(jnp.int32)
            b1 = b1 + (q0 + (CH - 1) >= pcu_k).astype(jnp.int32)
        p0 = q0 + sel(b0 + 1, [None] + cus[1:] + [cus[B]])
        contig = jnp.logical_and(b0 == b1, p0 % 8 == 0)

        @pl.when(contig)
        def _():
            p0a = pl.multiple_of(p0, 8)
            pltpu.async_copy(zbuf, out_hbm.at[pl.ds(p0a, CH)], sem_p)

        @pl.when(jnp.logical_not(contig))
        def _():
            for j in range(CH // LANES):
                qvj = (q0 + j * LANES) + iota
                b = jnp.zeros((LANES,), jnp.int32)
                for k in range(B):
                    b = b + (qvj >= ((k + 1) * L - cub[k])).astype(jnp.int32)
                pidx[c, pl.ds(j * LANES, LANES)] = (
                    plsc.load_gather(cu_v, [b + 1]) + qvj)
            pltpu.async_copy(zbuf, out_hbm.at[pidx.at[c]], sem_p)

    for c in range(NCH):
        write_pad(c)

    lds[0].wait()
    write_data(0)
    lds[1].wait()
    write_data(1)
    wait_write(0)
    lds[2] = load(2)
    wait_write(1)
    lds[3] = load(3)
    lds[2].wait()
    write_data(2)
    lds[3].wait()
    write_data(3)
    wait_write(2)
    wait_write(3)
    for _ in range(NCH):
        pltpu.make_async_copy(zbuf, out_hbm.at[pl.ds(0, CH)], sem_p).wait()


def kernel(flat, cu_seqlens):
    return _to_dense(flat, cu_seqlens.astype(jnp.int32)).reshape(B, L, D)
